# baseline (speedup 1.0000x reference)
import jax
import jax.numpy as jnp
from jax import lax
from jax.experimental import pallas as pl
from jax.experimental.pallas import tpu as pltpu
from jax.experimental.pallas import tpu_sc as plsc

H_DIM = 128
KSEL = 16
P = 64
D1 = 512
D2 = 256
EPS = 1e-5
NROW = 16384
NK = NROW * KSEL
B = 16
R0 = 512
R = 1024
NS0 = NROW // R0
NS = NROW // R
KP = KSEL * P


def _lrelu(x):
    return jnp.where(x >= 0, x, 0.01 * x)


def _sel_body(px_ref, pxc_ref, py_ref, pyc_ref, ones_ref, i64_ref, tie_ref,
              sel_ref):
    f32 = jnp.float32
    iota64 = jnp.broadcast_to(i64_ref[...], (KP, P))
    tiem = tie_ref[...]
    for b in range(B):
        pxr = px_ref[b:b + 1, :]
        pyr = py_ref[b:b + 1, :]
        pxc = pxc_ref[b]
        pyc = pyc_ref[b]
        dx = pxc - pxr
        dy = pyc - pyr
        d = jnp.sqrt(dx * dx + dy * dy)
        drep = jnp.tile(d, (KSEL, 1))
        dkb = jnp.concatenate(
            [jnp.broadcast_to(d[:, k:k + 1], (P, P)) for k in range(KSEL)],
            axis=0)
        m = (jnp.where(drep < dkb, 1.0, 0.0)
             + jnp.where(drep == dkb, tiem, 0.0))
        rkb = lax.dot(m, ones_ref[...], preferred_element_type=f32)
        off = (pl.program_id(0) * B + b) * P
        sel_ref[b] = rkb[:, 0:1].astype(jnp.int32) + off


def _mlp_body(x_ref, w1_ref, b1_ref, g1_ref, be1_ref, w2_ref, b2_ref,
              g2_ref, be2_ref, out_ref, y1_ref, y2_ref,
              s1_ref, q1_ref, s2_ref, q2_ref):
    t = pl.program_id(0)
    f32 = jnp.float32
    dot = lambda a, b: lax.dot(a, b, preferred_element_type=f32)
    nf = jnp.float32(NROW)

    @pl.when(t < NS0)
    def _phase0():
        i = t
        y = lax.dot(x_ref[...].astype(jnp.bfloat16), w1_ref[...],
                    preferred_element_type=f32)
        y = y + b1_ref[...]
        y1_ref[pl.ds(i * R0, R0), :] = y

        @pl.when(i == 0)
        def _():
            s1_ref[...] = jnp.zeros_like(s1_ref)
            q1_ref[...] = jnp.zeros_like(q1_ref)

        s1_ref[...] += jnp.sum(y, axis=0, keepdims=True)
        q1_ref[...] += jnp.sum(y * y, axis=0, keepdims=True)

    @pl.when((t >= NS0) & (t < NS0 + NS))
    def _phase1():
        i = t - NS0
        mean = s1_ref[...] / nf
        var = q1_ref[...] / nf - mean * mean
        scale = g1_ref[...] / jnp.sqrt(var + EPS)
        z = (y1_ref[pl.ds(i * R, R), :] - mean) * scale + be1_ref[...]
        z = _lrelu(z)
        y = dot(z, w2_ref[...])
        y = y + b2_ref[...]
        y2_ref[pl.ds(i * R, R), :] = y.astype(jnp.bfloat16)

        @pl.when(i == 0)
        def _():
            s2_ref[...] = jnp.zeros_like(s2_ref)
            q2_ref[...] = jnp.zeros_like(q2_ref)

        s2_ref[...] += jnp.sum(y, axis=0, keepdims=True)
        q2_ref[...] += jnp.sum(y * y, axis=0, keepdims=True)

    @pl.when(t >= NS0 + NS)
    def _phase2():
        i = t - NS0 - NS
        mean = s2_ref[...] / nf
        var = q2_ref[...] / nf - mean * mean
        scale = g2_ref[...] / jnp.sqrt(var + EPS)
        z = (y2_ref[pl.ds(i * R, R), :].astype(f32) - mean) * scale \
            + be2_ref[...]
        out_ref[...] = _lrelu(z)


def _make_sc_gather():
    info = plsc.get_sparse_core_info()
    NC, NS_, L = info.num_cores, info.num_subcores, info.num_lanes
    NW = NC * NS_
    per_w = NK // NW
    CH = 128
    NCH = per_w // CH
    mesh = plsc.VectorSubcoreMesh(core_axis_name="c", subcore_axis_name="s")

    @jax.jit
    def run(table, idx):
        import functools

        @functools.partial(
            pl.kernel, mesh=mesh,
            out_type=jax.ShapeDtypeStruct((NK, H_DIM), jnp.float32),
            scratch_types=[
                pltpu.VMEM((CH,), jnp.int32),
                pltpu.VMEM((CH, H_DIM), jnp.float32),
                pltpu.SemaphoreType.DMA,
            ],
        )
        def k(table_hbm, idx_hbm, out_hbm, idx_v, rows_v, sem):
            wid = lax.axis_index("s") * NC + lax.axis_index("c")
            base = wid * per_w

            def body(c, carry):
                off = base + c * CH
                pltpu.sync_copy(idx_hbm.at[pl.ds(off, CH)], idx_v)
                pltpu.async_copy(table_hbm.at[idx_v], rows_v, sem).wait()
                pltpu.sync_copy(rows_v, out_hbm.at[pl.ds(off, CH)])
                return carry

            lax.fori_loop(0, NCH, body, 0)

        return k(table, idx)

    return run


_sc_gather = _make_sc_gather()


def kernel(h_states, seq_start_end, last_pos, W1, b1, g1, be1, W2, b2, g2, be2):
    G = seq_start_end.shape[0]
    N = h_states.shape[0]

    px = last_pos[:, 0].reshape(G, P)
    py = last_pos[:, 1].reshape(G, P)
    pxc = px.reshape(G, P, 1)
    pyc = py.reshape(G, P, 1)

    ridx = jnp.arange(KP, dtype=jnp.int32)
    nidx = jnp.arange(P, dtype=jnp.int32)
    ones64 = jnp.ones((P, P), jnp.float32)
    i64 = nidx.astype(jnp.float32).reshape(1, P)
    tie2 = (nidx[None, :] < (ridx[:, None] // P)).astype(jnp.float32)

    sel = pl.pallas_call(
        _sel_body,
        grid=(G // B,),
        in_specs=[
            pl.BlockSpec((B, P), lambda i: (i, 0)),
            pl.BlockSpec((B, P, 1), lambda i: (i, 0, 0)),
            pl.BlockSpec((B, P), lambda i: (i, 0)),
            pl.BlockSpec((B, P, 1), lambda i: (i, 0, 0)),
            pl.BlockSpec((P, P), lambda i: (0, 0)),
            pl.BlockSpec((1, P), lambda i: (0, 0)),
            pl.BlockSpec((KP, P), lambda i: (0, 0)),
        ],
        out_specs=pl.BlockSpec((B, KP, 1), lambda i: (i, 0, 0)),
        out_shape=jax.ShapeDtypeStruct((G, KP, 1), jnp.int32),
    )(px, pxc, py, pyc, ones64, i64, tie2)

    idx = sel.reshape(G, KSEL, P).transpose(0, 2, 1).reshape(NK)

    x = _sc_gather(h_states, idx)
    x2 = x.reshape(N, KSEL * H_DIM)

    const2 = lambda t: (0, 0)
    out = pl.pallas_call(
        _mlp_body,
        grid=(NS0 + 2 * NS,),
        in_specs=[
            pl.BlockSpec((R0, KSEL * H_DIM), lambda t: (jnp.where(t < NS0, t, 0), 0)),
            pl.BlockSpec((KSEL * H_DIM, D1), const2),
            pl.BlockSpec((1, D1), const2),
            pl.BlockSpec((1, D1), const2),
            pl.BlockSpec((1, D1), const2),
            pl.BlockSpec((D1, D2), const2),
            pl.BlockSpec((1, D2), const2),
            pl.BlockSpec((1, D2), const2),
            pl.BlockSpec((1, D2), const2),
        ],
        out_specs=pl.BlockSpec(
            (R, D2), lambda t: (jnp.where(t >= NS0 + NS, t - NS0 - NS, 0), 0)),
        out_shape=jax.ShapeDtypeStruct((N, D2), jnp.float32),
        scratch_shapes=[
            pltpu.VMEM((NROW, D1), jnp.float32),
            pltpu.VMEM((NROW, D2), jnp.bfloat16),
            pltpu.VMEM((1, D1), jnp.float32),
            pltpu.VMEM((1, D1), jnp.float32),
            pltpu.VMEM((1, D2), jnp.float32),
            pltpu.VMEM((1, D2), jnp.float32),
        ],
    )(x2, W1.astype(jnp.bfloat16), b1.reshape(1, D1), g1.reshape(1, D1),
      be1.reshape(1, D1), W2, b2.reshape(1, D2), g2.reshape(1, D2),
      be2.reshape(1, D2))

    return out


# speedup vs baseline: 3.8560x; 3.8560x over previous
import jax
import jax.numpy as jnp
from jax import lax
from jax.experimental import pallas as pl
from jax.experimental.pallas import tpu as pltpu

H_DIM = 128
KSEL = 16
P = 64
D1 = 512
D2 = 256
EPS = 1e-5
NROW = 16384
B = 16
R = 1024
NS0 = 256 // B
NS = NROW // R


def _lrelu(x):
    return jnp.where(x >= 0, x, 0.01 * x)


def _body(px_ref, pxc_ref, py_ref, pyc_ref, h_ref, ones_ref, i64_ref,
          tie_ref, w1_ref, b1_ref, g1_ref, be1_ref,
          w2_ref, b2_ref, g2_ref, be2_ref,
          out_ref, y1_ref, y2_ref, s1_ref, q1_ref, s2_ref, q2_ref):
    t = pl.program_id(0)
    KP = KSEL * P
    f32 = jnp.float32
    dot = lambda a, b: lax.dot(a, b, preferred_element_type=f32)
    nf = jnp.float32(NROW)

    @pl.when(t < NS0)
    def _phase0():
        i = t
        bf16 = jnp.bfloat16
        iota64 = jnp.broadcast_to(i64_ref[...], (KP, P))
        tiem = tie_ref[...]
        x3 = []
        for b in range(B):
            pxr = px_ref[b:b + 1, :]
            pyr = py_ref[b:b + 1, :]
            pxc = pxc_ref[b]
            pyc = pyc_ref[b]
            dx = pxc - pxr
            dy = pyc - pyr
            d = jnp.sqrt(dx * dx + dy * dy)
            drep = jnp.tile(d, (KSEL, 1))
            dkb = jnp.concatenate(
                [jnp.broadcast_to(d[:, k:k + 1], (P, P))
                 for k in range(KSEL)], axis=0)
            m = (jnp.where(drep < dkb, 1.0, 0.0)
                 + jnp.where(drep == dkb, tiem, 0.0))
            rkb = dot(m, ones_ref[...])
            s = jnp.where(rkb == iota64, 1.0, 0.0)
            x3.append(dot(s, h_ref[b]))
        x = jnp.concatenate(
            [jnp.concatenate([x3[b][k * P:(k + 1) * P, :] for b in range(B)],
                             axis=0) for k in range(KSEL)], axis=1)
        y = lax.dot(x.astype(bf16), w1_ref[...], preferred_element_type=f32)
        y = y + b1_ref[...]
        y1_ref[pl.ds(i * (B * P), B * P), :] = y

        @pl.when(i == 0)
        def _():
            s1_ref[...] = jnp.zeros_like(s1_ref)
            q1_ref[...] = jnp.zeros_like(q1_ref)

        s1_ref[...] += jnp.sum(y, axis=0, keepdims=True)
        q1_ref[...] += jnp.sum(y * y, axis=0, keepdims=True)

    @pl.when((t >= NS0) & (t < NS0 + NS))
    def _phase1():
        i = t - NS0
        mean = s1_ref[...] / nf
        var = q1_ref[...] / nf - mean * mean
        scale = g1_ref[...] / jnp.sqrt(var + EPS)
        z = (y1_ref[pl.ds(i * R, R), :] - mean) * scale + be1_ref[...]
        z = _lrelu(z)
        y = dot(z, w2_ref[...])
        y = y + b2_ref[...]
        y2_ref[pl.ds(i * R, R), :] = y.astype(jnp.bfloat16)

        @pl.when(i == 0)
        def _():
            s2_ref[...] = jnp.zeros_like(s2_ref)
            q2_ref[...] = jnp.zeros_like(q2_ref)

        s2_ref[...] += jnp.sum(y, axis=0, keepdims=True)
        q2_ref[...] += jnp.sum(y * y, axis=0, keepdims=True)

    @pl.when(t >= NS0 + NS)
    def _phase2():
        i = t - NS0 - NS
        mean = s2_ref[...] / nf
        var = q2_ref[...] / nf - mean * mean
        scale = g2_ref[...] / jnp.sqrt(var + EPS)
        z = (y2_ref[pl.ds(i * R, R), :].astype(jnp.float32) - mean) * scale \
            + be2_ref[...]
        out_ref[...] = _lrelu(z)


def kernel(h_states, seq_start_end, last_pos, W1, b1, g1, be1, W2, b2, g2, be2):
    G = seq_start_end.shape[0]
    N = h_states.shape[0]

    px = last_pos[:, 0].reshape(G, P)
    py = last_pos[:, 1].reshape(G, P)
    pxc = px.reshape(G, P, 1)
    pyc = py.reshape(G, P, 1)
    h3 = h_states.reshape(G, P, H_DIM)

    KP = KSEL * P
    ridx = jnp.arange(KP, dtype=jnp.int32)
    nidx = jnp.arange(P, dtype=jnp.int32)
    ones64 = jnp.ones((P, P), jnp.float32)
    i64 = nidx.astype(jnp.float32).reshape(1, P)
    tie2 = (nidx[None, :] < (ridx[:, None] // P)).astype(jnp.float32)

    grp = lambda t: (jnp.where(t < NS0, t, 0), 0)
    grp3 = lambda t: (jnp.where(t < NS0, t, 0), 0, 0)
    const2 = lambda t: (0, 0)

    out = pl.pallas_call(
        _body,
        grid=(NS0 + 2 * NS,),
        in_specs=[
            pl.BlockSpec((B, P), grp),
            pl.BlockSpec((B, P, 1), grp3),
            pl.BlockSpec((B, P), grp),
            pl.BlockSpec((B, P, 1), grp3),
            pl.BlockSpec((B, P, H_DIM), grp3),
            pl.BlockSpec((P, P), const2),
            pl.BlockSpec((1, P), const2),
            pl.BlockSpec((KP, P), const2),
            pl.BlockSpec((KSEL * H_DIM, D1), const2),
            pl.BlockSpec((1, D1), const2),
            pl.BlockSpec((1, D1), const2),
            pl.BlockSpec((1, D1), const2),
            pl.BlockSpec((D1, D2), const2),
            pl.BlockSpec((1, D2), const2),
            pl.BlockSpec((1, D2), const2),
            pl.BlockSpec((1, D2), const2),
        ],
        out_specs=pl.BlockSpec(
            (R, D2), lambda t: (jnp.where(t >= NS0 + NS, t - NS0 - NS, 0), 0)),
        out_shape=jax.ShapeDtypeStruct((N, D2), jnp.float32),
        scratch_shapes=[
            pltpu.VMEM((NROW, D1), jnp.float32),
            pltpu.VMEM((NROW, D2), jnp.bfloat16),
            pltpu.VMEM((1, D1), jnp.float32),
            pltpu.VMEM((1, D1), jnp.float32),
            pltpu.VMEM((1, D2), jnp.float32),
            pltpu.VMEM((1, D2), jnp.float32),
        ],
    )(px, pxc, py, pyc, h3, ones64, i64, tie2,
      W1.astype(jnp.bfloat16), b1.reshape(1, D1), g1.reshape(1, D1),
      be1.reshape(1, D1), W2, b2.reshape(1, D2), g2.reshape(1, D2),
      be2.reshape(1, D2))

    return out
